# baseline (device time: 31702 ns/iter reference)
import jax
import jax.numpy as jnp
from jax import lax
from jax.experimental import pallas as pl
from jax.experimental.pallas import tpu as pltpu

T = 1024
D = 1024
V_LOCAL = 8192
HALF = T // 2
NC = 8
CH = HALF // NC


def kernel(ids, E):
    base_out = lax.axis_index("y") * V_LOCAL
    safe_ids = jnp.clip(ids - base_out, 0, V_LOCAL - 1).astype(jnp.int32)
    packed_ids = safe_ids[0::2] | (safe_ids[1::2] << 16)
    ids_v = ids.reshape(T, 1)

    def body(safe_s, ids_vref, e_ref, out_ref,
             gbuf, ysend, ybuf, xsend, xbuf,
             gsems, ys_sems, yr_sems, xs_sems, xr_sems):
        my_x = lax.axis_index("x")
        my_y = lax.axis_index("y")
        y_peer = (my_x, 1 - my_y)
        x_peer = (1 - my_x, my_y)
        base = my_y * V_LOCAL
        row0 = my_x * HALF
        other0 = (1 - my_x) * HALF

        barrier = pltpu.get_barrier_semaphore()
        for peer in (y_peer, x_peer):
            pl.semaphore_signal(
                barrier, inc=1, device_id=peer,
                device_id_type=pl.DeviceIdType.MESH,
            )
        pl.semaphore_wait(barrier, 2)

        def issue_chunk(c):
            pair0 = my_x * (HALF // 2) + c * (CH // 2)

            def issue(j, carry):
                w = safe_s[pair0 + j]
                i0 = w & 0xFFFF
                i1 = w >> 16
                d = c * CH + 2 * j
                pltpu.make_async_copy(
                    e_ref.at[pl.ds(i0, 1), :],
                    gbuf.at[pl.ds(d, 1), :],
                    gsems.at[c],
                ).start()
                pltpu.make_async_copy(
                    e_ref.at[pl.ds(i1, 1), :],
                    gbuf.at[pl.ds(d + 1, 1), :],
                    gsems.at[c],
                ).start()
                return carry

            lax.fori_loop(0, CH // 2, issue, 0, unroll=16)

        y_rdmas = {}
        x_rdmas = {}

        def send_y(c):
            def drain(j, carry):
                pltpu.make_async_copy(
                    e_ref.at[pl.ds(0, 1), :],
                    gbuf.at[pl.ds(0, 1), :],
                    gsems.at[c],
                ).wait()
                return carry

            lax.fori_loop(0, CH, drain, 0, unroll=16)
            sl = pl.ds(c * CH, CH)
            idxv = ids_vref[pl.ds(row0 + c * CH, CH), :] - base
            maskv = jnp.logical_and(idxv >= 0, idxv < V_LOCAL)
            ysend[sl, :] = jnp.where(maskv, gbuf[sl, :], 0.0).astype(
                jnp.bfloat16
            )
            r = pltpu.make_async_remote_copy(
                src_ref=ysend.at[sl, :],
                dst_ref=ybuf.at[sl, :],
                send_sem=ys_sems.at[c],
                recv_sem=yr_sems.at[c],
                device_id=y_peer,
                device_id_type=pl.DeviceIdType.MESH,
            )
            r.start()
            y_rdmas[c] = r

        def process_y(c):
            sl = pl.ds(c * CH, CH)
            y_rdmas[c].wait_recv()
            summ = ysend[sl, :] + ybuf[sl, :]
            xsend[sl, :] = summ
            out_ref[pl.ds(row0 + c * CH, CH), :] = summ.astype(jnp.float32)
            r = pltpu.make_async_remote_copy(
                src_ref=xsend.at[sl, :],
                dst_ref=xbuf.at[sl, :],
                send_sem=xs_sems.at[c],
                recv_sem=xr_sems.at[c],
                device_id=x_peer,
                device_id_type=pl.DeviceIdType.MESH,
            )
            r.start()
            x_rdmas[c] = r

        def process_x(c):
            sl = pl.ds(c * CH, CH)
            x_rdmas[c].wait_recv()
            out_ref[pl.ds(other0 + c * CH, CH), :] = (
                xbuf[sl, :].astype(jnp.float32)
            )

        issue_chunk(0)
        for c in range(NC):
            if c + 1 < NC:
                issue_chunk(c + 1)
            send_y(c)
            if c >= 1:
                process_y(c - 1)
            if c >= 2:
                process_x(c - 2)
        process_y(NC - 1)
        process_x(NC - 2)
        process_x(NC - 1)

        for c in range(NC):
            y_rdmas[c].wait_send()
            x_rdmas[c].wait_send()

    out = pl.pallas_call(
        body,
        out_shape=jax.ShapeDtypeStruct((T, D), jnp.float32),
        in_specs=[
            pl.BlockSpec(memory_space=pltpu.SMEM),
            pl.BlockSpec(memory_space=pltpu.VMEM),
            pl.BlockSpec(memory_space=pltpu.HBM),
        ],
        out_specs=pl.BlockSpec(memory_space=pltpu.VMEM),
        scratch_shapes=[
            pltpu.VMEM((HALF, D), jnp.float32),
            pltpu.VMEM((HALF, D), jnp.bfloat16),
            pltpu.VMEM((HALF, D), jnp.bfloat16),
            pltpu.VMEM((HALF, D), jnp.bfloat16),
            pltpu.VMEM((HALF, D), jnp.bfloat16),
            pltpu.SemaphoreType.DMA((NC,)),
            pltpu.SemaphoreType.DMA((NC,)),
            pltpu.SemaphoreType.DMA((NC,)),
            pltpu.SemaphoreType.DMA((NC,)),
            pltpu.SemaphoreType.DMA((NC,)),
        ],
        compiler_params=pltpu.CompilerParams(collective_id=0),
    )(packed_ids, ids_v, E)
    return out
